# baseline (speedup 1.0000x reference)
import functools

import jax
import jax.numpy as jnp
from jax import lax
from jax.experimental import pallas as pl
from jax.experimental.pallas import tpu as pltpu
from jax.experimental.pallas import tpu_sc as plsc

D = 64
B_TOTAL = 4096 * 200
SCALE = 8.0

NC = 2
NS = 16
NW = NC * NS
W = B_TOTAL // NW
C = 128
NCHUNK = W // C
NBUF = 4
LANES = 16
D_VECS = D // LANES


def _emb_kernel(x_hbm, table_hbm, out_hbm, idx_v, *bufs_and_sems):
  rows = bufs_and_sems[:NBUF]
  gsem = bufs_and_sems[NBUF:2 * NBUF]
  ssem = bufs_and_sems[2 * NBUF:3 * NBUF]

  wid = lax.axis_index("s") * NC + lax.axis_index("c")
  base = wid * W

  pltpu.sync_copy(x_hbm.at[pl.ds(base, W)], idx_v)

  def idx_slice(j):
    return idx_v.at[pl.ds(j * C, C)]

  def start_gather(j, b):
    pltpu.async_copy(table_hbm.at[idx_slice(j)], rows[b], gsem[b])

  def wait_gather(j, b):
    pltpu.make_async_copy(table_hbm.at[idx_slice(j)], rows[b], gsem[b]).wait()

  def start_store(j, b):
    pltpu.async_copy(rows[b], out_hbm.at[pl.ds(base + j * C, C)], ssem[b])

  def wait_store(j, b):
    pltpu.make_async_copy(
        rows[b], out_hbm.at[pl.ds(base + j * C, C)], ssem[b]).wait()

  for b in range(NBUF):
    start_gather(b, b)

  def outer(i, carry):
    j0 = i * NBUF
    for b in range(NBUF):
      j = j0 + b
      wait_gather(j, b)
      buf = rows[b]

      def scale_row(r, c2):
        for l in range(D_VECS):
          sl = (r, pl.ds(l * LANES, LANES))
          buf[sl] = buf[sl] * SCALE
        return c2

      lax.fori_loop(0, C, scale_row, 0)
      start_store(j, b)
      jn = j + NBUF

      @pl.when(jn < NCHUNK)
      def _():
        wait_store(j, b)
        start_gather(jn, b)

    return carry

  lax.fori_loop(0, NCHUNK // NBUF, outer, 0)

  for b in range(NBUF):
    wait_store(NCHUNK - NBUF + b, b)


@jax.jit
def _run(x_flat, table):
  mesh = plsc.VectorSubcoreMesh(core_axis_name="c", subcore_axis_name="s")
  scratch = [pltpu.VMEM((W,), jnp.int32)]
  scratch += [pltpu.VMEM((C, D), jnp.float32) for _ in range(NBUF)]
  scratch += [pltpu.SemaphoreType.DMA for _ in range(2 * NBUF)]
  fn = functools.partial(
      pl.kernel,
      mesh=mesh,
      out_type=jax.ShapeDtypeStruct((B_TOTAL, D), jnp.float32),
      scratch_types=scratch,
      compiler_params=pltpu.CompilerParams(use_tc_tiling_on_sc=False),
  )(_emb_kernel)
  return fn(x_flat, table)


def kernel(x, table):
  x_flat = x.reshape(-1).astype(jnp.int32)
  out = _run(x_flat, table)
  return out.reshape(x.shape + (D,))
